# baseline (speedup 1.0000x reference)
import jax
import jax.numpy as jnp
from jax import lax
from jax.experimental import pallas as pl
from jax.experimental.pallas import tpu as pltpu
from jax.experimental.pallas import tpu_sc as plsc

NU = 50000
D = 64
HD = 32
E = 800000
NL = 3
NC = 2
NS = 16
SUB = 80
MEGA = 25
NMEGA = 25
TPR = MEGA * NMEGA
LOOK = 3
NB = 6
CH = 3120
TAIL = NU - NS * CH
TROWS = NC * NU


_GDN = lax.GatherDimensionNumbers(
    offset_dims=(), collapsed_slice_dims=(0,), start_index_map=(0,))


def _lane_broadcast(v16, i):
    idx = jnp.full((16, 1), i, jnp.int32)
    return lax.gather(v16, idx, _GDN, (1,),
                      mode=lax.GatherScatterMode.PROMISE_IN_BOUNDS)


def _sc_all_body(tabs_in, src_sel, didx_sel, w_hbm, out,
                 sidx, didx, wbuf, rows, zbuf, acc, gsems, ssems, stgsem):
    cid = lax.axis_index("c")
    sid = lax.axis_index("s")

    def _zrow(r, _):
        zbuf[r, pl.ds(0, 16)] = jnp.zeros((16,), jnp.float32)
        zbuf[r, pl.ds(16, 16)] = jnp.zeros((16,), jnp.float32)
        return 0
    lax.fori_loop(0, SUB, _zrow, 0)

    def _pass(p, _):
        dir_ = lax.rem(p, 2)
        base = jnp.where(p < 2, (1 - dir_) * TROWS,
                         (p - 1 - 2 * dir_) * TROWS)

        def _zcopy(k, _):
            pltpu.async_copy(zbuf, acc.at[pl.ds(sid * CH + k * SUB, SUB)],
                             stgsem)
            return 0
        lax.fori_loop(0, CH // SUB, _zcopy, 0)

        @pl.when(sid == 0)
        def _ztail():
            pltpu.async_copy(zbuf, acc.at[pl.ds(NS * CH, TAIL)], stgsem)

        def _zdrain(k, _):
            pltpu.make_async_copy(
                zbuf, acc.at[pl.ds(sid * CH + k * SUB, SUB)], stgsem).wait()
            return 0
        lax.fori_loop(0, CH // SUB, _zdrain, 0)

        @pl.when(sid == 0)
        def _ztdrain():
            pltpu.make_async_copy(zbuf, acc.at[pl.ds(NS * CH, TAIL)],
                                  stgsem).wait()

        plsc.subcore_barrier()

        def _stage_start(m, slot):
            g = sid * NMEGA + m
            pltpu.async_copy(src_sel.at[dir_, cid, g], sidx.at[slot], stgsem)
            pltpu.async_copy(didx_sel.at[dir_, g], didx.at[slot], stgsem)
            pltpu.async_copy(w_hbm.at[g], wbuf.at[slot], stgsem)

        def _stage_finish(m, slot):
            g = sid * NMEGA + m
            pltpu.make_async_copy(src_sel.at[dir_, cid, g], sidx.at[slot],
                                  stgsem).wait()
            pltpu.make_async_copy(didx_sel.at[dir_, g], didx.at[slot],
                                  stgsem).wait()
            pltpu.make_async_copy(w_hbm.at[g], wbuf.at[slot], stgsem).wait()
            bvec = jnp.full((16,), base, jnp.int32)

            def _adj(lr, _):
                for g2 in range(SUB // 16):
                    sl = pl.ds(g2 * 16, 16)
                    sidx[slot, lr, sl] = sidx[slot, lr, sl] + bvec
                return 0
            lax.fori_loop(0, MEGA, _adj, 0)

        _stage_start(0, 0)
        _stage_finish(0, 0)
        _stage_start(1, 1)

        def _scatter_wait(b):
            pltpu.make_async_copy(rows.at[b], acc.at[didx.at[0, 0]],
                                  ssems.at[b]).wait()

        def _run_pipeline(src_tab):
            def _gather_desc(t, b):
                m = t // MEGA
                lr = t - m * MEGA
                return pltpu.make_async_copy(
                    src_tab.at[sidx.at[lax.rem(m, 2), lr]], rows.at[b],
                    gsems.at[b])

            def _step(t, _):
                @pl.when(t < TPR)
                def _issue():
                    b = lax.rem(t, NB)

                    @pl.when(t >= NB)
                    def _reuse():
                        _scatter_wait(b)

                    m = t // MEGA
                    lr = t - m * MEGA

                    @pl.when(jnp.logical_and(lr == 0, t > 0))
                    def _enter():
                        _stage_finish(m, lax.rem(m, 2))

                    _gather_desc(t, b).start()

                @pl.when(t >= LOOK)
                def _consume():
                    r = t - LOOK
                    b = lax.rem(r, NB)
                    m = r // MEGA
                    lr = r - m * MEGA
                    _gather_desc(r, b).wait()
                    for g in range(SUB // 16):
                        w16 = wbuf[lax.rem(m, 2), lr, pl.ds(g * 16, 16)]
                        for i in range(16):
                            e = g * 16 + i
                            wb = _lane_broadcast(w16, i)
                            rows[b, e, pl.ds(0, 16)] = (
                                rows[b, e, pl.ds(0, 16)] * wb)
                            rows[b, e, pl.ds(16, 16)] = (
                                rows[b, e, pl.ds(16, 16)] * wb)
                    pltpu.async_copy(rows.at[b],
                                     acc.at[didx.at[lax.rem(m, 2), lr]],
                                     ssems.at[b], add=True)

                    @pl.when(jnp.logical_and(lr == 0, r > 0))
                    def _cross():
                        @pl.when(m + 1 < NMEGA)
                        def _pref():
                            _stage_start(m + 1, lax.rem(m + 1, 2))

                return 0

            lax.fori_loop(0, TPR + LOOK, _step, 0)

        @pl.when(p < 2)
        def _from_inputs():
            _run_pipeline(tabs_in)

        @pl.when(p >= 2)
        def _from_layers():
            _run_pipeline(out)

        for b in range(NB):
            _scatter_wait(b)

        plsc.subcore_barrier()

        obase = p * TROWS + cid * NU
        pltpu.sync_copy(
            acc.at[pl.ds(sid * CH, CH)],
            out.at[pl.ds(obase + sid * CH, CH), :])

        @pl.when(sid == 0)
        def _dtail():
            pltpu.sync_copy(
                acc.at[pl.ds(NS * CH, TAIL)],
                out.at[pl.ds(obase + NS * CH, TAIL), :])

        plsc.subcore_barrier()
        return 0

    lax.fori_loop(0, 2 * NL, _pass, 0)


_sc_all = pl.kernel(
    _sc_all_body,
    out_type=jax.ShapeDtypeStruct((2 * NL * TROWS, HD), jnp.float32),
    mesh=plsc.VectorSubcoreMesh(core_axis_name="c", subcore_axis_name="s",
                                num_cores=NC, num_subcores=NS),
    scratch_types=[
        pltpu.VMEM((2, MEGA, SUB), jnp.int32),
        pltpu.VMEM((2, MEGA, SUB), jnp.int32),
        pltpu.VMEM((2, MEGA, SUB), jnp.float32),
        pltpu.VMEM((NB, SUB, HD), jnp.float32),
        pltpu.VMEM((SUB, HD), jnp.float32),
        pltpu.VMEM_SHARED((NU, HD), jnp.float32),
        pltpu.SemaphoreType.DMA((NB,)),
        pltpu.SemaphoreType.DMA((NB,)),
        pltpu.SemaphoreType.DMA,
    ],
    compiler_params=pltpu.CompilerParams(use_tc_tiling_on_sc=False),
)


def _avg_body(e0, l1a, l1b, l2a, l2b, l3a, l3b, out):
    q = jnp.float32(1.0 / (NL + 1))
    out[:, 0:HD] = (e0[:, 0:HD] + l1a[...] + l2a[...] + l3a[...]) * q
    out[:, HD:D] = (e0[:, HD:D] + l1b[...] + l2b[...] + l3b[...]) * q


_AVG_R = 2000
_AVG_GRID = NU // _AVG_R


def _avg(e0, s1, s2, s3):
    lo = pl.BlockSpec((_AVG_R, HD), lambda j: (j, 0))
    hi = pl.BlockSpec((_AVG_R, HD), lambda j: (j + _AVG_GRID, 0))
    return pl.pallas_call(
        _avg_body,
        grid=(_AVG_GRID,),
        in_specs=[pl.BlockSpec((_AVG_R, D), lambda j: (j, 0)),
                  lo, hi, lo, hi, lo, hi],
        out_specs=pl.BlockSpec((_AVG_R, D), lambda j: (j, 0)),
        out_shape=jax.ShapeDtypeStruct((NU, D), jnp.float32),
    )(e0, s1, s1, s2, s2, s3, s3)


def kernel(user_embedding, item_embedding, edge_user, edge_item, edge_weight):
    tu0 = jnp.concatenate([user_embedding[:, :HD], user_embedding[:, HD:]], 0)
    ti0 = jnp.concatenate([item_embedding[:, :HD], item_embedding[:, HD:]], 0)
    tabs_in = jnp.concatenate([tu0, ti0], 0)
    eu3 = edge_user.reshape(NS * NMEGA, MEGA, SUB)
    ei3 = edge_item.reshape(NS * NMEGA, MEGA, SUB)
    w3 = edge_weight.reshape(NS * NMEGA, MEGA, SUB)
    src_sel = jnp.stack([jnp.stack([ei3, ei3 + NU]),
                         jnp.stack([eu3, eu3 + NU])])
    didx_sel = jnp.stack([eu3, ei3])

    flat = _sc_all(tabs_in, src_sel, didx_sel, w3)
    layers = flat.reshape(2 * NL, TROWS, HD)
    embed_user = _avg(user_embedding, layers[0], layers[2], layers[4])
    embed_item = _avg(item_embedding, layers[1], layers[3], layers[5])
    return (embed_user, embed_item)


# speedup vs baseline: 1.5431x; 1.5431x over previous
import jax
import jax.numpy as jnp
from jax import lax
from jax.experimental import pallas as pl
from jax.experimental.pallas import tpu as pltpu
from jax.experimental.pallas import tpu_sc as plsc

NU = 50000
D = 64
HD = 32
E = 800000
NL = 3
NC = 2
NS = 16
SUB = 80
MEGA = 25
NMEGA = 25
TPR = MEGA * NMEGA
LOOK = 3
NB = 6
CH = 3120
TAIL = NU - NS * CH


_GDN = lax.GatherDimensionNumbers(
    offset_dims=(), collapsed_slice_dims=(0,), start_index_map=(0,))


def _lane_broadcast(v16, i):
    idx = jnp.full((16, 1), i, jnp.int32)
    return lax.gather(v16, idx, _GDN, (1,),
                      mode=lax.GatherScatterMode.PROMISE_IN_BOUNDS)


def _sc_pass_body(table, src3, didx_hbm, w_hbm, out,
                  sidx, didx, wbuf, rows, acc, gsems, ssems, stgsem):
    cid = lax.axis_index("c")
    sid = lax.axis_index("s")

    def _zrow(r, _):
        rows[0, r, pl.ds(0, 16)] = jnp.zeros((16,), jnp.float32)
        rows[0, r, pl.ds(16, 16)] = jnp.zeros((16,), jnp.float32)
        return 0
    lax.fori_loop(0, SUB, _zrow, 0)

    def _zcopy(k, _):
        pltpu.async_copy(rows.at[0], acc.at[pl.ds(sid * CH + k * SUB, SUB)],
                         stgsem)
        return 0
    lax.fori_loop(0, CH // SUB, _zcopy, 0)

    @pl.when(sid == 0)
    def _ztail():
        pltpu.async_copy(rows.at[0], acc.at[pl.ds(NS * CH, TAIL)], stgsem)

    def _zdrain(k, _):
        pltpu.make_async_copy(
            rows.at[0], acc.at[pl.ds(sid * CH + k * SUB, SUB)], stgsem).wait()
        return 0
    lax.fori_loop(0, CH // SUB, _zdrain, 0)

    @pl.when(sid == 0)
    def _ztdrain():
        pltpu.make_async_copy(rows.at[0], acc.at[pl.ds(NS * CH, TAIL)],
                              stgsem).wait()

    plsc.subcore_barrier()

    def _stage_start(m, slot):
        g = sid * NMEGA + m
        pltpu.async_copy(src3.at[cid, g], sidx.at[slot], stgsem)
        pltpu.async_copy(didx_hbm.at[g], didx.at[slot], stgsem)
        pltpu.async_copy(w_hbm.at[g], wbuf.at[slot], stgsem)

    def _stage_wait(m, slot):
        g = sid * NMEGA + m
        pltpu.make_async_copy(src3.at[cid, g], sidx.at[slot], stgsem).wait()
        pltpu.make_async_copy(didx_hbm.at[g], didx.at[slot], stgsem).wait()
        pltpu.make_async_copy(w_hbm.at[g], wbuf.at[slot], stgsem).wait()

    _stage_start(0, 0)
    _stage_wait(0, 0)
    _stage_start(1, 1)

    def _gather_desc(r, b):
        m = r // MEGA
        lr = r - m * MEGA
        return pltpu.make_async_copy(
            table.at[sidx.at[lax.rem(m, 2), lr]], rows.at[b], gsems.at[b])

    def _scatter_wait(b):
        pltpu.make_async_copy(rows.at[b], acc.at[didx.at[0, 0]],
                              ssems.at[b]).wait()

    def _step(t, _):
        @pl.when(t < TPR)
        def _issue():
            b = lax.rem(t, NB)

            @pl.when(t >= NB)
            def _reuse():
                _scatter_wait(b)

            m = t // MEGA
            lr = t - m * MEGA

            @pl.when(jnp.logical_and(lr == 0, t > 0))
            def _enter():
                _stage_wait(m, lax.rem(m, 2))

            _gather_desc(t, b).start()

        @pl.when(t >= LOOK)
        def _consume():
            r = t - LOOK
            b = lax.rem(r, NB)
            m = r // MEGA
            lr = r - m * MEGA
            _gather_desc(r, b).wait()
            for g in range(SUB // 16):
                w16 = wbuf[lax.rem(m, 2), lr, pl.ds(g * 16, 16)]
                for i in range(16):
                    e = g * 16 + i
                    wb = _lane_broadcast(w16, i)
                    rows[b, e, pl.ds(0, 16)] = rows[b, e, pl.ds(0, 16)] * wb
                    rows[b, e, pl.ds(16, 16)] = rows[b, e, pl.ds(16, 16)] * wb
            pltpu.async_copy(rows.at[b], acc.at[didx.at[lax.rem(m, 2), lr]],
                             ssems.at[b], add=True)

            @pl.when(jnp.logical_and(lr == 0, r > 0))
            def _cross():
                @pl.when(m + 1 < NMEGA)
                def _pref():
                    _stage_start(m + 1, lax.rem(m + 1, 2))

        return 0

    lax.fori_loop(0, TPR + LOOK, _step, 0)

    for b in range(NB):
        _scatter_wait(b)

    plsc.subcore_barrier()

    pltpu.sync_copy(
        acc.at[pl.ds(sid * CH, CH)],
        out.at[pl.ds(cid * NU + sid * CH, CH), :])

    @pl.when(sid == 0)
    def _dtail():
        pltpu.sync_copy(
            acc.at[pl.ds(NS * CH, TAIL)],
            out.at[pl.ds(cid * NU + NS * CH, TAIL), :])


_sc_pass = pl.kernel(
    _sc_pass_body,
    out_type=jax.ShapeDtypeStruct((NC * NU, HD), jnp.float32),
    mesh=plsc.VectorSubcoreMesh(core_axis_name="c", subcore_axis_name="s",
                                num_cores=NC, num_subcores=NS),
    scratch_types=[
        pltpu.VMEM((2, MEGA, SUB), jnp.int32),
        pltpu.VMEM((2, MEGA, SUB), jnp.int32),
        pltpu.VMEM((2, MEGA, SUB), jnp.float32),
        pltpu.VMEM((NB, SUB, HD), jnp.float32),
        pltpu.VMEM_SHARED((NU, HD), jnp.float32),
        pltpu.SemaphoreType.DMA((NB,)),
        pltpu.SemaphoreType.DMA((NB,)),
        pltpu.SemaphoreType.DMA,
    ],
    compiler_params=pltpu.CompilerParams(use_tc_tiling_on_sc=False),
)


def _avg_body(e0, l1a, l1b, l2a, l2b, l3a, l3b, out):
    q = jnp.float32(1.0 / (NL + 1))
    out[:, 0:HD] = (e0[:, 0:HD] + l1a[...] + l2a[...] + l3a[...]) * q
    out[:, HD:D] = (e0[:, HD:D] + l1b[...] + l2b[...] + l3b[...]) * q


_AVG_R = 2000
_AVG_GRID = NU // _AVG_R


def _avg(e0, s1, s2, s3):
    lo = pl.BlockSpec((_AVG_R, HD), lambda j: (j, 0))
    hi = pl.BlockSpec((_AVG_R, HD), lambda j: (j + _AVG_GRID, 0))
    return pl.pallas_call(
        _avg_body,
        grid=(_AVG_GRID,),
        in_specs=[pl.BlockSpec((_AVG_R, D), lambda j: (j, 0)),
                  lo, hi, lo, hi, lo, hi],
        out_specs=pl.BlockSpec((_AVG_R, D), lambda j: (j, 0)),
        out_shape=jax.ShapeDtypeStruct((NU, D), jnp.float32),
    )(e0, s1, s1, s2, s2, s3, s3)


def kernel(user_embedding, item_embedding, edge_user, edge_item, edge_weight):
    tu0 = jnp.concatenate([user_embedding[:, :HD], user_embedding[:, HD:]], 0)
    ti0 = jnp.concatenate([item_embedding[:, :HD], item_embedding[:, HD:]], 0)
    eu3 = edge_user.reshape(NS * NMEGA, MEGA, SUB)
    ei3 = edge_item.reshape(NS * NMEGA, MEGA, SUB)
    w3 = edge_weight.reshape(NS * NMEGA, MEGA, SUB)
    src_item = jnp.stack([ei3, ei3 + NU])
    src_user = jnp.stack([eu3, eu3 + NU])

    ti, tu = ti0, tu0
    us, its = [], []
    for _ in range(NL):
        u_new = _sc_pass(ti, src_item, eu3, w3)
        i_new = _sc_pass(tu, src_user, ei3, w3)
        us.append(u_new)
        its.append(i_new)
        tu, ti = u_new, i_new

    embed_user = _avg(user_embedding, us[0], us[1], us[2])
    embed_item = _avg(item_embedding, its[0], its[1], its[2])
    return (embed_user, embed_item)


# speedup vs baseline: 1.7106x; 1.1085x over previous
import jax
import jax.numpy as jnp
from jax import lax
from jax.experimental import pallas as pl
from jax.experimental.pallas import tpu as pltpu
from jax.experimental.pallas import tpu_sc as plsc

NU = 50000
D = 64
HD = 32
E = 800000
NL = 3
NC = 2
NS = 16
SUB = 80
MEGA = 25
NMEGA = 25
TPR = MEGA * NMEGA
LOOK = 5
NB = 7
CH = 3120
TAIL = NU - NS * CH


_GDN = lax.GatherDimensionNumbers(
    offset_dims=(), collapsed_slice_dims=(0,), start_index_map=(0,))


def _lane_broadcast(v16, i):
    idx = jnp.full((16, 1), i, jnp.int32)
    return lax.gather(v16, idx, _GDN, (1,),
                      mode=lax.GatherScatterMode.PROMISE_IN_BOUNDS)


def _sc_pass_body(table, src3, didx_hbm, w_hbm, out,
                  sidx, didx, wbuf, rows, acc, gsems, ssems, stgsem):
    cid = lax.axis_index("c")
    sid = lax.axis_index("s")

    def _zrow(r, _):
        rows[0, r, pl.ds(0, 16)] = jnp.zeros((16,), jnp.float32)
        rows[0, r, pl.ds(16, 16)] = jnp.zeros((16,), jnp.float32)
        return 0
    lax.fori_loop(0, SUB, _zrow, 0)

    def _zcopy(k, _):
        pltpu.async_copy(rows.at[0], acc.at[pl.ds(sid * CH + k * SUB, SUB)],
                         stgsem)
        return 0
    lax.fori_loop(0, CH // SUB, _zcopy, 0)

    @pl.when(sid == 0)
    def _ztail():
        pltpu.async_copy(rows.at[0], acc.at[pl.ds(NS * CH, TAIL)], stgsem)

    def _zdrain(k, _):
        pltpu.make_async_copy(
            rows.at[0], acc.at[pl.ds(sid * CH + k * SUB, SUB)], stgsem).wait()
        return 0
    lax.fori_loop(0, CH // SUB, _zdrain, 0)

    @pl.when(sid == 0)
    def _ztdrain():
        pltpu.make_async_copy(rows.at[0], acc.at[pl.ds(NS * CH, TAIL)],
                              stgsem).wait()

    plsc.subcore_barrier()

    def _stage_start(m, slot):
        g = sid * NMEGA + m
        pltpu.async_copy(src3.at[cid, g], sidx.at[slot], stgsem)
        pltpu.async_copy(didx_hbm.at[g], didx.at[slot], stgsem)
        pltpu.async_copy(w_hbm.at[g], wbuf.at[slot], stgsem)

    def _stage_wait(m, slot):
        g = sid * NMEGA + m
        pltpu.make_async_copy(src3.at[cid, g], sidx.at[slot], stgsem).wait()
        pltpu.make_async_copy(didx_hbm.at[g], didx.at[slot], stgsem).wait()
        pltpu.make_async_copy(w_hbm.at[g], wbuf.at[slot], stgsem).wait()

    _stage_start(0, 0)
    _stage_wait(0, 0)
    _stage_start(1, 1)

    def _gather_desc(r, b):
        m = r // MEGA
        lr = r - m * MEGA
        return pltpu.make_async_copy(
            table.at[sidx.at[lax.rem(m, 2), lr]], rows.at[b], gsems.at[b])

    def _scatter_wait(b):
        pltpu.make_async_copy(rows.at[b], acc.at[didx.at[0, 0]],
                              ssems.at[b]).wait()

    def _step(t, _):
        @pl.when(t < TPR)
        def _issue():
            b = lax.rem(t, NB)

            @pl.when(t >= NB)
            def _reuse():
                _scatter_wait(b)

            m = t // MEGA
            lr = t - m * MEGA

            @pl.when(jnp.logical_and(lr == 0, t > 0))
            def _enter():
                _stage_wait(m, lax.rem(m, 2))

            _gather_desc(t, b).start()

        @pl.when(t >= LOOK)
        def _consume():
            r = t - LOOK
            b = lax.rem(r, NB)
            m = r // MEGA
            lr = r - m * MEGA
            _gather_desc(r, b).wait()
            for g in range(SUB // 16):
                w16 = wbuf[lax.rem(m, 2), lr, pl.ds(g * 16, 16)]
                for i in range(16):
                    e = g * 16 + i
                    wb = _lane_broadcast(w16, i)
                    rows[b, e, pl.ds(0, 16)] = rows[b, e, pl.ds(0, 16)] * wb
                    rows[b, e, pl.ds(16, 16)] = rows[b, e, pl.ds(16, 16)] * wb
            pltpu.async_copy(rows.at[b], acc.at[didx.at[lax.rem(m, 2), lr]],
                             ssems.at[b], add=True)

            @pl.when(jnp.logical_and(lr == 0, r > 0))
            def _cross():
                @pl.when(m + 1 < NMEGA)
                def _pref():
                    _stage_start(m + 1, lax.rem(m + 1, 2))

        return 0

    lax.fori_loop(0, TPR + LOOK, _step, 0)

    for b in range(NB):
        _scatter_wait(b)

    plsc.subcore_barrier()

    pltpu.sync_copy(
        acc.at[pl.ds(sid * CH, CH)],
        out.at[pl.ds(cid * NU + sid * CH, CH), :])

    @pl.when(sid == 0)
    def _dtail():
        pltpu.sync_copy(
            acc.at[pl.ds(NS * CH, TAIL)],
            out.at[pl.ds(cid * NU + NS * CH, TAIL), :])


_sc_pass = pl.kernel(
    _sc_pass_body,
    out_type=jax.ShapeDtypeStruct((NC * NU, HD), jnp.float32),
    mesh=plsc.VectorSubcoreMesh(core_axis_name="c", subcore_axis_name="s",
                                num_cores=NC, num_subcores=NS),
    scratch_types=[
        pltpu.VMEM((2, MEGA, SUB), jnp.int32),
        pltpu.VMEM((2, MEGA, SUB), jnp.int32),
        pltpu.VMEM((2, MEGA, SUB), jnp.float32),
        pltpu.VMEM((NB, SUB, HD), jnp.float32),
        pltpu.VMEM_SHARED((NU, HD), jnp.float32),
        pltpu.SemaphoreType.DMA((NB,)),
        pltpu.SemaphoreType.DMA((NB,)),
        pltpu.SemaphoreType.DMA,
    ],
    compiler_params=pltpu.CompilerParams(use_tc_tiling_on_sc=False),
)


def _avg_body(e0, l1a, l1b, l2a, l2b, l3a, l3b, out):
    q = jnp.float32(1.0 / (NL + 1))
    out[:, 0:HD] = (e0[:, 0:HD] + l1a[...] + l2a[...] + l3a[...]) * q
    out[:, HD:D] = (e0[:, HD:D] + l1b[...] + l2b[...] + l3b[...]) * q


_AVG_R = 2000
_AVG_GRID = NU // _AVG_R


def _avg(e0, s1, s2, s3):
    lo = pl.BlockSpec((_AVG_R, HD), lambda j: (j, 0))
    hi = pl.BlockSpec((_AVG_R, HD), lambda j: (j + _AVG_GRID, 0))
    return pl.pallas_call(
        _avg_body,
        grid=(_AVG_GRID,),
        in_specs=[pl.BlockSpec((_AVG_R, D), lambda j: (j, 0)),
                  lo, hi, lo, hi, lo, hi],
        out_specs=pl.BlockSpec((_AVG_R, D), lambda j: (j, 0)),
        out_shape=jax.ShapeDtypeStruct((NU, D), jnp.float32),
    )(e0, s1, s1, s2, s2, s3, s3)


def kernel(user_embedding, item_embedding, edge_user, edge_item, edge_weight):
    tu0 = jnp.concatenate([user_embedding[:, :HD], user_embedding[:, HD:]], 0)
    ti0 = jnp.concatenate([item_embedding[:, :HD], item_embedding[:, HD:]], 0)
    eu3 = edge_user.reshape(NS * NMEGA, MEGA, SUB)
    ei3 = edge_item.reshape(NS * NMEGA, MEGA, SUB)
    w3 = edge_weight.reshape(NS * NMEGA, MEGA, SUB)
    src_item = jnp.stack([ei3, ei3 + NU])
    src_user = jnp.stack([eu3, eu3 + NU])

    ti, tu = ti0, tu0
    us, its = [], []
    for _ in range(NL):
        u_new = _sc_pass(ti, src_item, eu3, w3)
        i_new = _sc_pass(tu, src_user, ei3, w3)
        us.append(u_new)
        its.append(i_new)
        tu, ti = u_new, i_new

    embed_user = _avg(user_embedding, us[0], us[1], us[2])
    embed_item = _avg(item_embedding, its[0], its[1], its[2])
    return (embed_user, embed_item)
